# baseline (speedup 1.0000x reference)
import functools

import jax
import jax.numpy as jnp
from jax import lax
from jax.experimental import pallas as pl
from jax.experimental.pallas import tpu as pltpu
from jax.experimental.pallas import tpu_sc as plsc


def _make_gather(B, D, C):
    info = plsc.get_sparse_core_info()
    nc, ns = info.num_cores, info.num_subcores
    nw = nc * ns
    n_per_w = B // nw
    n_chunks = n_per_w // C
    mesh = plsc.VectorSubcoreMesh(core_axis_name="c", subcore_axis_name="s")

    @functools.partial(
        pl.kernel,
        out_type=jax.ShapeDtypeStruct((B, D), jnp.float32),
        mesh=mesh,
        scratch_types=[
            pltpu.VMEM((C,), jnp.int32),
            pltpu.VMEM((C, D), jnp.float32),
            pltpu.SemaphoreType.DMA,
        ],
        compiler_params=pltpu.CompilerParams(use_tc_tiling_on_sc=False),
    )
    def k(idx_hbm, table_hbm, out_hbm, idx_v, rows_v, sem):
        wid = lax.axis_index("s") * nc + lax.axis_index("c")

        @pl.loop(0, n_chunks)
        def _(j):
            base = wid * n_per_w + j * C
            pltpu.sync_copy(idx_hbm.at[pl.ds(base, C)], idx_v)
            pltpu.async_copy(table_hbm.at[idx_v], rows_v, sem).wait()
            pltpu.sync_copy(rows_v, out_hbm.at[pl.ds(base, C)])

    return k


def kernel(inputs, table):
    D = table.shape[1]
    B = inputs.shape[0] * inputs.shape[1]
    idx = inputs.reshape(B).astype(jnp.int32)
    out = _make_gather(B, D, 800)(idx, table)
    return out.reshape(inputs.shape + (D,)), inputs


# speedup vs baseline: 1.2285x; 1.2285x over previous
import functools

import jax
import jax.numpy as jnp
from jax import lax
from jax.experimental import pallas as pl
from jax.experimental.pallas import tpu as pltpu
from jax.experimental.pallas import tpu_sc as plsc


def _make_gather(B, D, C):
    info = plsc.get_sparse_core_info()
    nc, ns = info.num_cores, info.num_subcores
    nw = nc * ns
    n_per_w = B // nw
    n_chunks = n_per_w // C
    mesh = plsc.VectorSubcoreMesh(core_axis_name="c", subcore_axis_name="s")

    @functools.partial(
        pl.kernel,
        out_type=jax.ShapeDtypeStruct((B, 2 * D), jnp.float32),
        mesh=mesh,
        scratch_types=[
            pltpu.VMEM((C,), jnp.int32),
            pltpu.VMEM((C, 2 * D), jnp.float32),
            pltpu.SemaphoreType.DMA,
        ],
        compiler_params=pltpu.CompilerParams(use_tc_tiling_on_sc=True),
    )
    def k(idx_hbm, table_hbm, out_hbm, idx_v, rows_v, sem):
        wid = lax.axis_index("s") * nc + lax.axis_index("c")

        @pl.loop(0, n_chunks)
        def _(j):
            base = wid * n_per_w + j * C
            pltpu.sync_copy(idx_hbm.at[pl.ds(base, C)], idx_v)
            pltpu.async_copy(table_hbm.at[idx_v], rows_v, sem).wait()
            pltpu.sync_copy(rows_v, out_hbm.at[pl.ds(base, C)])

    return k


def kernel(inputs, table):
    D = table.shape[1]
    B = inputs.shape[0] * inputs.shape[1]
    idx = inputs.reshape(B).astype(jnp.int32)
    tpad = jnp.pad(table, ((0, 0), (0, D)))
    out = _make_gather(B, D, 800)(idx, tpad)
    return out[:, :D].reshape(inputs.shape + (D,)), inputs
